# Initial kernel scaffold; baseline (speedup 1.0000x reference)
import jax
import jax.numpy as jnp
from jax.experimental import pallas as pl


def kernel(x, table):
    raise NotImplementedError("write your pallas kernel here")


# speedup vs baseline: 7.6623x; 7.6623x over previous
import jax
import jax.numpy as jnp
from jax import lax
from jax.experimental import pallas as pl
from jax.experimental.pallas import tpu as pltpu
from jax.experimental.pallas import tpu_sc as plsc

_B, _L, _V, _E = 4096, 200, 32128, 128
_NC, _NS = 2, 16
_NW = _NC * _NS
_BPW = _B // _NW
_IPW = _BPW * _L
_NL = 16
_EV = _E // _NL
_C0 = 128
_C1 = _L - _C0


def _body(x_hbm, table_hbm, out_hbm, idx_v, rows_v, out_stage, sem):
    wid = lax.axis_index("s") * _NC + lax.axis_index("c")
    pltpu.sync_copy(x_hbm.at[pl.ds(wid * _IPW, _IPW)], idx_v)

    def gather(b, rows):
        off = pl.multiple_of(b * _L, 8)
        c0 = pltpu.async_copy(
            table_hbm.at[idx_v.at[pl.ds(off, _C0)]], rows.at[pl.ds(0, _C0)], sem)
        c1 = pltpu.async_copy(
            table_hbm.at[idx_v.at[pl.ds(off + _C0, _C1)]],
            rows.at[pl.ds(_C0, _C1)], sem)
        c0.wait()
        c1.wait()

    def reduce_store(rows, b):
        def jbody(j, acc):
            return tuple(acc[k] + rows[j, pl.ds(k * _NL, _NL)]
                         for k in range(_EV))
        acc = tuple(rows[0, pl.ds(k * _NL, _NL)] for k in range(_EV))
        acc = lax.fori_loop(1, _L, jbody, acc)
        for k in range(_EV):
            out_stage[b, pl.ds(k * _NL, _NL)] = acc[k]

    def bloop(b, carry):
        gather(b, rows_v)
        reduce_store(rows_v, b)
        return carry

    lax.fori_loop(0, _BPW, bloop, 0)
    pltpu.sync_copy(out_stage, out_hbm.at[pl.ds(wid * _BPW, _BPW)])


def kernel(x, table):
    xf = x.reshape(-1)
    mesh = plsc.VectorSubcoreMesh(core_axis_name="c", subcore_axis_name="s")
    f = pl.kernel(
        _body,
        out_type=jax.ShapeDtypeStruct((_B, _E), jnp.float32),
        mesh=mesh,
        scratch_types=[
            pltpu.VMEM((_IPW,), jnp.int32),
            pltpu.VMEM((_L, _E), jnp.float32),
            pltpu.VMEM((_BPW, _E), jnp.float32),
            pltpu.SemaphoreType.DMA,
        ],
    )
    return f(xf, table)


# speedup vs baseline: 13.4657x; 1.7574x over previous
import jax
import jax.numpy as jnp
from jax import lax
from jax.experimental import pallas as pl
from jax.experimental.pallas import tpu as pltpu
from jax.experimental.pallas import tpu_sc as plsc

_B, _L, _V, _E = 4096, 200, 32128, 128
_NC, _NS = 2, 16
_NW = _NC * _NS
_BPW = _B // _NW
_IPW = _BPW * _L
_NL = 16
_EV = _E // _NL
_C0 = 128
_C1 = _L - _C0


def _body(x_hbm, table_hbm, out_hbm, idx_v, rows0, rows1, out_stage,
          sem0, sem1):
    wid = lax.axis_index("s") * _NC + lax.axis_index("c")
    pltpu.sync_copy(x_hbm.at[pl.ds(wid * _IPW, _IPW)], idx_v)

    def start(b, rows, sem):
        off = pl.multiple_of(b * _L, 8)
        pltpu.async_copy(
            table_hbm.at[idx_v.at[pl.ds(off, _C0)]], rows.at[pl.ds(0, _C0)], sem)
        pltpu.async_copy(
            table_hbm.at[idx_v.at[pl.ds(off + _C0, _C1)]],
            rows.at[pl.ds(_C0, _C1)], sem)

    def wait(rows, sem):
        pltpu.make_async_copy(table_hbm.at[pl.ds(0, _L)], rows, sem).wait()

    def reduce_store(rows, b):
        def jbody(j, acc):
            return tuple(acc[k] + rows[j, pl.ds(k * _NL, _NL)]
                         for k in range(_EV))
        acc = tuple(rows[0, pl.ds(k * _NL, _NL)] for k in range(_EV))
        acc = lax.fori_loop(1, _L, jbody, acc)
        for k in range(_EV):
            out_stage[b, pl.ds(k * _NL, _NL)] = acc[k]

    start(0, rows0, sem0)
    pairs = _BPW // 2

    def pair(i, carry):
        b0 = 2 * i
        start(b0 + 1, rows1, sem1)
        wait(rows0, sem0)
        reduce_store(rows0, b0)

        @pl.when(i < pairs - 1)
        def _():
            start(b0 + 2, rows0, sem0)

        wait(rows1, sem1)
        reduce_store(rows1, b0 + 1)
        return carry

    lax.fori_loop(0, pairs, pair, 0)
    pltpu.sync_copy(out_stage, out_hbm.at[pl.ds(wid * _BPW, _BPW)])


def kernel(x, table):
    xf = x.reshape(-1)
    mesh = plsc.VectorSubcoreMesh(core_axis_name="c", subcore_axis_name="s")
    f = pl.kernel(
        _body,
        out_type=jax.ShapeDtypeStruct((_B, _E), jnp.float32),
        mesh=mesh,
        scratch_types=[
            pltpu.VMEM((_IPW,), jnp.int32),
            pltpu.VMEM((_L, _E), jnp.float32),
            pltpu.VMEM((_L, _E), jnp.float32),
            pltpu.VMEM((_BPW, _E), jnp.float32),
            pltpu.SemaphoreType.DMA,
            pltpu.SemaphoreType.DMA,
        ],
    )
    return f(xf, table)
